# speedup vs baseline: 1.9707x; 1.9707x over previous
import jax
import jax.numpy as jnp
from jax import lax
from jax.experimental import pallas as pl
from jax.experimental.pallas import tpu as pltpu


def _stats_body(x_ref, s1_ref, s2_ref, acc1_ref, acc2_ref):
    j = pl.program_id(1)

    @pl.when(j == 0)
    def _init():
        acc1_ref[...] = jnp.zeros_like(acc1_ref)
        acc2_ref[...] = jnp.zeros_like(acc2_ref)

    x = x_ref[...]
    acc1_ref[...] += jnp.sum(x, axis=0)
    acc2_ref[...] += jnp.sum(x * x, axis=0)

    @pl.when(j == pl.num_programs(1) - 1)
    def _flush():
        s1_ref[...] = jnp.sum(acc1_ref[...], axis=1, keepdims=True)[None]
        s2_ref[...] = jnp.sum(acc2_ref[...], axis=1, keepdims=True)[None]


def _make_conv_body(img_blk, C, H, W, OC, inv_count, eps):
    HW = H * W
    KKC = 9 * C

    def _body(x_ref, s1_ref, s2_ref, g_ref, b_ref, w_ref, o_ref, p_ref):
        s1 = s1_ref[0] + s1_ref[1]
        s2 = s2_ref[0] + s2_ref[1]
        mean = s1 * inv_count
        var = s2 * inv_count - mean * mean
        scale = g_ref[...] * lax.rsqrt(var + eps)
        shift = b_ref[...] - mean * scale

        lane = lax.broadcasted_iota(jnp.int32, (1, HW), 1)
        wpos = lane % W
        m_first = wpos != 0
        m_last = wpos != (W - 1)

        wmat = w_ref[...]
        zero = jnp.bfloat16(0)
        for b in range(img_blk):
            y = jnp.maximum(x_ref[b] * scale + shift, 0.0).astype(jnp.bfloat16)
            for kh in range(3):
                for kw in range(3):
                    d = (kh - 1) * W + (kw - 1)
                    if d == 0:
                        t = y
                    elif d > 0:
                        t = jnp.concatenate(
                            [y[:, d:], jnp.zeros((C, d), jnp.bfloat16)], axis=1)
                    else:
                        t = jnp.concatenate(
                            [jnp.zeros((C, -d), jnp.bfloat16), y[:, :HW + d]],
                            axis=1)
                    if kw == 0:
                        t = jnp.where(m_first, t, zero)
                    elif kw == 2:
                        t = jnp.where(m_last, t, zero)
                    k = kh * 3 + kw
                    p_ref[k * C:(k + 1) * C, :] = t
            o_ref[b] = jnp.dot(wmat, p_ref[...],
                               preferred_element_type=jnp.float32)

    return _body


def kernel(x_nchw, gamma, beta, weight_oihw, *, eps=1e-5):
    N, C, H, W = x_nchw.shape
    OC, Cin, KH, KW = weight_oihw.shape
    assert Cin == C and KH == 3 and KW == 3
    HW = H * W
    KKC = 9 * C

    img_blk = 8 if N % 16 == 0 else 1
    steps = N // img_blk
    half = steps // 2

    x_slab = x_nchw.reshape(N, C, HW).astype(jnp.float32)
    w_mat = (jnp.transpose(weight_oihw, (0, 2, 3, 1))
             .reshape(OC, KKC).astype(jnp.bfloat16))
    gamma2d = gamma.reshape(C, 1).astype(jnp.float32)
    beta2d = beta.reshape(C, 1).astype(jnp.float32)

    s1, s2 = pl.pallas_call(
        _stats_body,
        out_shape=(jax.ShapeDtypeStruct((2, C, 1), jnp.float32),
                   jax.ShapeDtypeStruct((2, C, 1), jnp.float32)),
        grid=(2, half),
        in_specs=[pl.BlockSpec((img_blk, C, HW),
                               lambda i, j: (i * half + j, 0, 0))],
        out_specs=(pl.BlockSpec((1, C, 1), lambda i, j: (i, 0, 0)),
                   pl.BlockSpec((1, C, 1), lambda i, j: (i, 0, 0))),
        scratch_shapes=[pltpu.VMEM((C, HW), jnp.float32),
                        pltpu.VMEM((C, HW), jnp.float32)],
        compiler_params=pltpu.CompilerParams(
            dimension_semantics=("parallel", "arbitrary")),
    )(x_slab)

    conv_body = _make_conv_body(img_blk, C, H, W, OC,
                                1.0 / float(N * HW), eps)
    out = pl.pallas_call(
        conv_body,
        out_shape=jax.ShapeDtypeStruct((N, OC, HW), jnp.float32),
        grid=(steps,),
        in_specs=[pl.BlockSpec((img_blk, C, HW), lambda i: (i, 0, 0)),
                  pl.BlockSpec((2, C, 1), lambda i: (0, 0, 0)),
                  pl.BlockSpec((2, C, 1), lambda i: (0, 0, 0)),
                  pl.BlockSpec((C, 1), lambda i: (0, 0)),
                  pl.BlockSpec((C, 1), lambda i: (0, 0)),
                  pl.BlockSpec((OC, KKC), lambda i: (0, 0))],
        out_specs=pl.BlockSpec((img_blk, OC, HW), lambda i: (i, 0, 0)),
        scratch_shapes=[pltpu.VMEM((KKC, HW), jnp.bfloat16)],
        compiler_params=pltpu.CompilerParams(
            dimension_semantics=("parallel",)),
    )(x_slab, s1, s2, gamma2d, beta2d, w_mat)

    return out.reshape(N, OC, H, W)
